# Initial kernel scaffold; baseline (speedup 1.0000x reference)
import jax
import jax.numpy as jnp
from jax.experimental import pallas as pl


def kernel(input_ids, game_state_table, position_table, ln_gamma, ln_beta):
    raise NotImplementedError("write your pallas kernel here")


# speedup vs baseline: 4.6088x; 4.6088x over previous
import functools

import jax
import jax.numpy as jnp
from jax.experimental import pallas as pl

STATE_SIZE = 6
HIDDEN = 256
SEQ = 200
PAD_TOKEN = 1
EPS = 1e-12

BB = 16


def _combo_body(game_ref, pos_ref, gamma_ref, beta_ref, out_ref):
    x = game_ref[...][:, None, :] + pos_ref[...][None, :, :]
    mean = jnp.mean(x, axis=-1, keepdims=True)
    xc = x - mean
    var = jnp.mean(xc * xc, axis=-1, keepdims=True)
    y = xc * jax.lax.rsqrt(var + EPS)
    out_ref[...] = y * gamma_ref[...][None, :, :] + beta_ref[...][None, :, :]


def _gather_body(ids_ref, ids3_ref, combo_ref, out_ref, mask_ref):
    ids = ids_ref[...]
    mask_ref[...] = (ids != PAD_TOKEN).astype(jnp.int32)
    ids3 = ids3_ref[...]
    combo = combo_ref[...]
    acc = jnp.broadcast_to(combo[0][None], (BB, SEQ, HIDDEN))
    for t in range(1, STATE_SIZE):
        acc = jnp.where(ids3 == t, combo[t][None], acc)
    out_ref[...] = acc


def kernel(input_ids, game_state_table, position_table, ln_gamma, ln_beta):
    batch, seq = input_ids.shape
    ids = input_ids.astype(jnp.int32)

    combo = pl.pallas_call(
        _combo_body,
        out_shape=jax.ShapeDtypeStruct((STATE_SIZE, SEQ, HIDDEN), jnp.float32),
        in_specs=[
            pl.BlockSpec((STATE_SIZE, HIDDEN), lambda: (0, 0)),
            pl.BlockSpec((SEQ, HIDDEN), lambda: (0, 0)),
            pl.BlockSpec((1, HIDDEN), lambda: (0, 0)),
            pl.BlockSpec((1, HIDDEN), lambda: (0, 0)),
        ],
        out_specs=pl.BlockSpec((STATE_SIZE, SEQ, HIDDEN), lambda: (0, 0, 0)),
    )(
        game_state_table,
        position_table[:SEQ],
        ln_gamma.reshape(1, HIDDEN),
        ln_beta.reshape(1, HIDDEN),
    )

    grid = (batch // BB,)
    emb, mask = pl.pallas_call(
        _gather_body,
        grid=grid,
        out_shape=(
            jax.ShapeDtypeStruct((batch, seq, HIDDEN), jnp.float32),
            jax.ShapeDtypeStruct((batch, seq), jnp.int32),
        ),
        in_specs=[
            pl.BlockSpec((BB, seq), lambda i: (i, 0)),
            pl.BlockSpec((BB, seq, 1), lambda i: (i, 0, 0)),
            pl.BlockSpec((STATE_SIZE, SEQ, HIDDEN), lambda i: (0, 0, 0)),
        ],
        out_specs=(
            pl.BlockSpec((BB, seq, HIDDEN), lambda i: (i, 0, 0)),
            pl.BlockSpec((BB, seq), lambda i: (i, 0)),
        ),
    )(ids, ids.reshape(batch, seq, 1), combo)

    return emb, mask


# speedup vs baseline: 4.6539x; 1.0098x over previous
import functools

import jax
import jax.numpy as jnp
from jax.experimental import pallas as pl

STATE_SIZE = 6
HIDDEN = 256
SEQ = 200
PAD_TOKEN = 1
EPS = 1e-12

BB = 32


def _combo_body(game_ref, pos_ref, gamma_ref, beta_ref, out_ref):
    x = game_ref[...][:, None, :] + pos_ref[...][None, :, :]
    mean = jnp.mean(x, axis=-1, keepdims=True)
    xc = x - mean
    var = jnp.mean(xc * xc, axis=-1, keepdims=True)
    y = xc * jax.lax.rsqrt(var + EPS)
    out_ref[...] = y * gamma_ref[...][None, :, :] + beta_ref[...][None, :, :]


def _gather_body(ids_ref, ids3_ref, combo_ref, out_ref, mask_ref):
    ids = ids_ref[...]
    mask_ref[...] = (ids != PAD_TOKEN).astype(jnp.int32)
    ids3 = ids3_ref[...]
    combo = combo_ref[...]
    acc = jnp.broadcast_to(combo[0][None], (BB, SEQ, HIDDEN))
    for t in range(1, STATE_SIZE):
        acc = jnp.where(ids3 == t, combo[t][None], acc)
    out_ref[...] = acc


def kernel(input_ids, game_state_table, position_table, ln_gamma, ln_beta):
    batch, seq = input_ids.shape
    ids = input_ids.astype(jnp.int32)

    combo = pl.pallas_call(
        _combo_body,
        out_shape=jax.ShapeDtypeStruct((STATE_SIZE, SEQ, HIDDEN), jnp.float32),
        in_specs=[
            pl.BlockSpec((STATE_SIZE, HIDDEN), lambda: (0, 0)),
            pl.BlockSpec((SEQ, HIDDEN), lambda: (0, 0)),
            pl.BlockSpec((1, HIDDEN), lambda: (0, 0)),
            pl.BlockSpec((1, HIDDEN), lambda: (0, 0)),
        ],
        out_specs=pl.BlockSpec((STATE_SIZE, SEQ, HIDDEN), lambda: (0, 0, 0)),
    )(
        game_state_table,
        position_table[:SEQ],
        ln_gamma.reshape(1, HIDDEN),
        ln_beta.reshape(1, HIDDEN),
    )

    grid = (batch // BB,)
    emb, mask = pl.pallas_call(
        _gather_body,
        grid=grid,
        out_shape=(
            jax.ShapeDtypeStruct((batch, seq, HIDDEN), jnp.float32),
            jax.ShapeDtypeStruct((batch, seq), jnp.int32),
        ),
        in_specs=[
            pl.BlockSpec((BB, seq), lambda i: (i, 0)),
            pl.BlockSpec((BB, seq, 1), lambda i: (i, 0, 0)),
            pl.BlockSpec((STATE_SIZE, SEQ, HIDDEN), lambda i: (0, 0, 0)),
        ],
        out_specs=(
            pl.BlockSpec((BB, seq, HIDDEN), lambda i: (i, 0, 0)),
            pl.BlockSpec((BB, seq), lambda i: (i, 0)),
        ),
    )(ids, ids.reshape(batch, seq, 1), combo)

    return emb, mask


# speedup vs baseline: 5.4112x; 1.1627x over previous
import jax
import jax.numpy as jnp
from jax import lax
from jax.experimental import pallas as pl
from jax.experimental.pallas import tpu as pltpu
from jax.experimental.pallas import tpu_sc as plsc

STATE_SIZE = 6
HIDDEN = 256
SEQ = 200
PAD_TOKEN = 1
EPS = 1e-12

NC = 2
NS = 16
NW = NC * NS
CHUNK = 128
PREP_BB = 256


def _combo_body(game_ref, pos_ref, gamma_ref, beta_ref, out_ref):
    x = game_ref[...][:, None, :] + pos_ref[...][None, :, :]
    mean = jnp.mean(x, axis=-1, keepdims=True)
    xc = x - mean
    var = jnp.mean(xc * xc, axis=-1, keepdims=True)
    y = xc * jax.lax.rsqrt(var + EPS)
    out_ref[...] = y * gamma_ref[...][None, :, :] + beta_ref[...][None, :, :]


def _prep_body(ids_ref, mask_ref, idx_ref):
    ids = ids_ref[...]
    mask_ref[...] = (ids != PAD_TOKEN).astype(jnp.int32)
    s = lax.broadcasted_iota(jnp.int32, ids.shape, 1)
    idx_ref[...] = ids * SEQ + s


def _sc_gather_body(combo_hbm, idx_hbm, out_hbm, idx_v, buf0, buf1, gsem0, gsem1):
    n_chunks = idx_v.shape[0]
    rows_per_w = n_chunks * CHUNK
    wid = lax.axis_index("s") * NC + lax.axis_index("c")
    rows_base = wid * rows_per_w
    pltpu.sync_copy(idx_hbm.at[wid], idx_v)
    pltpu.async_copy(combo_hbm.at[idx_v.at[0]], buf0, gsem0)

    @pl.loop(0, n_chunks, step=2)
    def _chunks(jj):
        for b in range(2):
            j = jj + b
            buf_cur, sem_cur = (buf0, gsem0) if b == 0 else (buf1, gsem1)
            buf_nxt, sem_nxt = (buf1, gsem1) if b == 0 else (buf0, gsem0)

            @pl.when(j < n_chunks - 1)
            def _start_next():
                pltpu.async_copy(combo_hbm.at[idx_v.at[j + 1]], buf_nxt, sem_nxt)

            pltpu.make_async_copy(combo_hbm.at[idx_v.at[j]], buf_cur, sem_cur).wait()
            pltpu.sync_copy(buf_cur, out_hbm.at[pl.ds(rows_base + j * CHUNK, CHUNK)])


def kernel(input_ids, game_state_table, position_table, ln_gamma, ln_beta):
    batch, seq = input_ids.shape
    ids = input_ids.astype(jnp.int32)
    total_rows = batch * seq
    n_chunks = total_rows // (NW * CHUNK)

    combo = pl.pallas_call(
        _combo_body,
        out_shape=jax.ShapeDtypeStruct((STATE_SIZE, SEQ, HIDDEN), jnp.float32),
        in_specs=[
            pl.BlockSpec((STATE_SIZE, HIDDEN), lambda: (0, 0)),
            pl.BlockSpec((SEQ, HIDDEN), lambda: (0, 0)),
            pl.BlockSpec((1, HIDDEN), lambda: (0, 0)),
            pl.BlockSpec((1, HIDDEN), lambda: (0, 0)),
        ],
        out_specs=pl.BlockSpec((STATE_SIZE, SEQ, HIDDEN), lambda: (0, 0, 0)),
    )(
        game_state_table,
        position_table[:SEQ],
        ln_gamma.reshape(1, HIDDEN),
        ln_beta.reshape(1, HIDDEN),
    )
    combo_flat = combo.reshape(STATE_SIZE * SEQ, HIDDEN)

    mask, flat_idx = pl.pallas_call(
        _prep_body,
        grid=(batch // PREP_BB,),
        out_shape=(
            jax.ShapeDtypeStruct((batch, seq), jnp.int32),
            jax.ShapeDtypeStruct((batch, seq), jnp.int32),
        ),
        in_specs=[pl.BlockSpec((PREP_BB, seq), lambda i: (i, 0))],
        out_specs=(
            pl.BlockSpec((PREP_BB, seq), lambda i: (i, 0)),
            pl.BlockSpec((PREP_BB, seq), lambda i: (i, 0)),
        ),
    )(ids)

    idx3 = flat_idx.reshape(NW, n_chunks, CHUNK)

    sc_gather = pl.kernel(
        _sc_gather_body,
        out_type=jax.ShapeDtypeStruct((total_rows, HIDDEN), jnp.float32),
        mesh=plsc.VectorSubcoreMesh(core_axis_name="c", subcore_axis_name="s"),
        scratch_types=[
            pltpu.VMEM((n_chunks, CHUNK), jnp.int32),
            pltpu.VMEM((CHUNK, HIDDEN), jnp.float32),
            pltpu.VMEM((CHUNK, HIDDEN), jnp.float32),
            pltpu.SemaphoreType.DMA,
            pltpu.SemaphoreType.DMA,
        ],
    )
    out_flat = sc_gather(combo_flat, idx3)

    return out_flat.reshape(batch, seq, HIDDEN), mask


# speedup vs baseline: 6.6341x; 1.2260x over previous
import jax
import jax.numpy as jnp
from jax import lax
from jax.experimental import pallas as pl
from jax.experimental.pallas import tpu as pltpu
from jax.experimental.pallas import tpu_sc as plsc

STATE_SIZE = 6
HIDDEN = 256
SEQ = 200
PAD_TOKEN = 1
EPS = 1e-12

NC = 2
NS = 16
NW = NC * NS
CHUNK = 128
PREP_BB = 256
REP = 8


def _combo_body(game_ref, pos_ref, gamma_ref, beta_ref, out_ref):
    x = game_ref[...][:, None, :] + pos_ref[...][None, :, :]
    mean = jnp.mean(x, axis=-1, keepdims=True)
    xc = x - mean
    var = jnp.mean(xc * xc, axis=-1, keepdims=True)
    y = xc * jax.lax.rsqrt(var + EPS)
    out_ref[...] = (y * gamma_ref[...][None, :, :] + beta_ref[...][None, :, :])[None]


def _prep_body(ids_ref, mask_ref, idx_ref):
    ids = ids_ref[...]
    mask_ref[...] = (ids != PAD_TOKEN).astype(jnp.int32)
    s = lax.broadcasted_iota(jnp.int32, ids.shape, 1)
    row = lax.broadcasted_iota(jnp.int32, ids.shape, 0) + pl.program_id(0) * PREP_BB
    rep = (row // 128) % REP
    idx_ref[...] = ids * SEQ + s + rep * (STATE_SIZE * SEQ)


def _sc_gather_body(combo_hbm, idx_hbm, out_hbm, idx_v, buf0, buf1,
                    gsem0, gsem1):
    n_chunks = idx_v.shape[0]
    rows_per_w = n_chunks * CHUNK
    wid = lax.axis_index("s") * NC + lax.axis_index("c")
    rows_base = wid * rows_per_w
    pltpu.sync_copy(idx_hbm.at[wid], idx_v)
    pltpu.async_copy(combo_hbm.at[idx_v.at[0]], buf0, gsem0)

    @pl.loop(0, n_chunks, step=2)
    def _chunks(jj):
        for b in range(2):
            j = jj + b
            buf_cur, sem_cur = (buf0, gsem0) if b == 0 else (buf1, gsem1)
            buf_nxt, sem_nxt = (buf1, gsem1) if b == 0 else (buf0, gsem0)

            @pl.when(j < n_chunks - 1)
            def _start_next():
                pltpu.async_copy(combo_hbm.at[idx_v.at[j + 1]], buf_nxt, sem_nxt)

            pltpu.make_async_copy(combo_hbm.at[idx_v.at[j]], buf_cur, sem_cur).wait()
            pltpu.sync_copy(buf_cur, out_hbm.at[pl.ds(rows_base + j * CHUNK, CHUNK)])


def kernel(input_ids, game_state_table, position_table, ln_gamma, ln_beta):
    batch, seq = input_ids.shape
    ids = input_ids.astype(jnp.int32)
    total_rows = batch * seq
    n_chunks = total_rows // (NW * CHUNK)

    combo = pl.pallas_call(
        _combo_body,
        grid=(REP,),
        out_shape=jax.ShapeDtypeStruct((REP, STATE_SIZE, SEQ, HIDDEN), jnp.float32),
        in_specs=[
            pl.BlockSpec((STATE_SIZE, HIDDEN), lambda r: (0, 0)),
            pl.BlockSpec((SEQ, HIDDEN), lambda r: (0, 0)),
            pl.BlockSpec((1, HIDDEN), lambda r: (0, 0)),
            pl.BlockSpec((1, HIDDEN), lambda r: (0, 0)),
        ],
        out_specs=pl.BlockSpec((1, STATE_SIZE, SEQ, HIDDEN), lambda r: (r, 0, 0, 0)),
    )(
        game_state_table,
        position_table[:SEQ],
        ln_gamma.reshape(1, HIDDEN),
        ln_beta.reshape(1, HIDDEN),
    )
    combo_flat = combo.reshape(REP * STATE_SIZE * SEQ, HIDDEN)

    mask, flat_idx = pl.pallas_call(
        _prep_body,
        grid=(batch // PREP_BB,),
        out_shape=(
            jax.ShapeDtypeStruct((batch, seq), jnp.int32),
            jax.ShapeDtypeStruct((batch, seq), jnp.int32),
        ),
        in_specs=[pl.BlockSpec((PREP_BB, seq), lambda i: (i, 0))],
        out_specs=(
            pl.BlockSpec((PREP_BB, seq), lambda i: (i, 0)),
            pl.BlockSpec((PREP_BB, seq), lambda i: (i, 0)),
        ),
    )(ids)

    idx3 = flat_idx.reshape(NW, n_chunks, CHUNK)

    sc_gather = pl.kernel(
        _sc_gather_body,
        out_type=jax.ShapeDtypeStruct((total_rows, HIDDEN), jnp.float32),
        mesh=plsc.VectorSubcoreMesh(core_axis_name="c", subcore_axis_name="s"),
        scratch_types=[
            pltpu.VMEM((n_chunks, CHUNK), jnp.int32),
            pltpu.VMEM((CHUNK, HIDDEN), jnp.float32),
            pltpu.VMEM((CHUNK, HIDDEN), jnp.float32),
            pltpu.SemaphoreType.DMA,
            pltpu.SemaphoreType.DMA,
        ],
    )
    out_flat = sc_gather(combo_flat, idx3)

    return out_flat.reshape(batch, seq, HIDDEN), mask
